# speedup vs baseline: 1.0034x; 1.0034x over previous
import jax
import jax.numpy as jnp
import numpy as np
from jax.experimental import pallas as pl
from jax.experimental.pallas import tpu as pltpu

B = 2
N_PER = 5000
N = B * N_PER
R = 4
SAMPLE = 2048
KNN = 32


def _ffn(x, W0, b0, W1, b1):
    return jax.nn.relu(x @ W0 + b0) @ W1 + b1


def _bn(x, g, b):
    mu = jnp.mean(x, axis=0)
    var = jnp.var(x, axis=0)
    return (x - mu) / jnp.sqrt(var + 1e-5) * g + b


def _fps_one(pos_b, npoints):
    n = pos_b.shape[0]

    def step(carry, _):
        dists, last = carry
        d = jnp.sum((pos_b - pos_b[last]) ** 2, axis=1)
        dists = jnp.minimum(dists, d)
        nxt = jnp.argmax(dists).astype(jnp.int32)
        return (dists, nxt), last

    (_, _), idxs = jax.lax.scan(
        step, (jnp.full((n,), 1e10, jnp.float32), jnp.int32(0)), None,
        length=npoints)
    return idxs


def kernel(pos, edge_index, params):
    src = edge_index[0]
    dst = edge_index[1]
    deg = jnp.clip(jnp.zeros((N,), jnp.float32).at[dst].add(1.0), 1.0, None)
    feats = [pos]
    f = pos
    for r in range(R):
        f = jnp.zeros((N, 3), jnp.float32).at[dst].add(f[src]) / deg[:, None]
        feats.append(f)

    hs = [_ffn(feats[r], params['inc%d_W0' % r], params['inc%d_b0' % r],
               params['inc%d_W1' % r], params['inc%d_b1' % r])
          for r in range(R + 1)]
    x_ = _ffn(jax.nn.relu(jnp.concatenate(hs, axis=1)),
              params['proj_W0'], params['proj_b0'],
              params['proj_W1'], params['proj_b1'])
    xb = x_.reshape(B, N_PER, 64)
    gate = jax.nn.softmax(xb @ params['gate_W'] + params['gate_b'], axis=1)
    gfeat = jnp.broadcast_to(
        jnp.sum(gate * xb, axis=1, keepdims=True), (B, N_PER, 64))
    p = _bn(jax.nn.relu(pos @ params['pm_W0'] + params['pm_b0']),
            params['bn0_g'], params['bn0_b'])
    p = _bn(jax.nn.relu(p @ params['pm_W1'] + params['pm_b1']),
            params['bn1_g'], params['bn1_b'])
    p = (p @ params['pm_W2'] + params['pm_b2']).reshape(B, N_PER, 64)
    x = jnp.concatenate([xb, p, gfeat], axis=2)

    posb = pos.reshape(B, N_PER, 3)
    idxs = jax.vmap(lambda pb: _fps_one(pb, SAMPLE))(posb)

    sp = jnp.take_along_axis(posb, idxs[:, :, None], axis=1)
    sx = jnp.take_along_axis(x, idxs[:, :, None], axis=1)
    d2 = jnp.sum((sp[:, :, None, :] - sp[:, None, :, :]) ** 2, axis=-1)
    _, nbr = jax.lax.top_k(-d2, KNN)

    t = sx @ params['theta_W']
    pphi = sx @ params['phi_W']
    t_nb = jax.vmap(lambda ts, nb: ts[nb])(t, nbr)
    tmax = jnp.max(t_nb, axis=2)
    x_out = (tmax - t + pphi + params['theta_b'] + params['phi_b']
             ).reshape(B * SAMPLE, 256)

    offs = (jnp.arange(B) * SAMPLE)[:, None]
    new_src = (nbr.reshape(B, -1) + offs).reshape(-1)
    new_dst = (jnp.tile(jnp.repeat(jnp.arange(SAMPLE), KNN)[None, :],
                        (B, 1)) + offs).reshape(-1)
    return (x_out, jnp.stack([new_src, new_dst]))


# speedup vs baseline: 1.6019x; 1.5964x over previous
import jax
import jax.numpy as jnp
import numpy as np
from jax.experimental import pallas as pl
from jax.experimental.pallas import tpu as pltpu

B = 2
N_PER = 5000
N = B * N_PER
R = 4
SAMPLE = 2048
KNN = 32


def _ffn(x, W0, b0, W1, b1):
    return jax.nn.relu(x @ W0 + b0) @ W1 + b1


def _bn(x, g, b):
    mu = jnp.mean(x, axis=0)
    var = jnp.var(x, axis=0)
    return (x - mu) / jnp.sqrt(var + 1e-5) * g + b


ROWS = 40
NPAD = ROWS * 128
OROWS = SAMPLE // 128
BIG_I = np.int32(2**30)


def _fps_body(pos_ref, out_ref, dists_ref, coord_ref):
    flat = (jax.lax.broadcasted_iota(jnp.int32, (ROWS, 128), 0) * 128
            + jax.lax.broadcasted_iota(jnp.int32, (ROWS, 128), 1))
    oflat = (jax.lax.broadcasted_iota(jnp.int32, (OROWS, 128), 0) * 128
             + jax.lax.broadcasted_iota(jnp.int32, (OROWS, 128), 1))
    valid = flat < N_PER
    ninf = jnp.float32(-jnp.inf)
    for b in range(B):
        dists_ref[b] = jnp.where(valid, jnp.float32(1e10), ninf)
        out_ref[b] = jnp.zeros((OROWS, 128), jnp.int32)
        for c in range(3):
            coord_ref[b, c] = pos_ref[b, c, 0:1, 0:1]

    def step(k, _):
        for b in range(B):
            px = pos_ref[b, 0]
            py = pos_ref[b, 1]
            pz = pos_ref[b, 2]
            dx = px - coord_ref[b, 0]
            dy = py - coord_ref[b, 1]
            dz = pz - coord_ref[b, 2]
            d = (dx * dx + dy * dy) + dz * dz
            nd = jnp.minimum(dists_ref[b], d)
            dists_ref[b] = nd
            m = jnp.max(nd, axis=(0, 1), keepdims=True)
            idx = jnp.min(jnp.where(nd == m, flat, BIG_I), axis=(0, 1),
                          keepdims=True)
            win = flat == idx
            out_ref[b] = jnp.where(oflat == k, idx[0, 0], out_ref[b])
            coord_ref[b, 0] = jnp.max(jnp.where(win, px, ninf), axis=(0, 1),
                                      keepdims=True)
            coord_ref[b, 1] = jnp.max(jnp.where(win, py, ninf), axis=(0, 1),
                                      keepdims=True)
            coord_ref[b, 2] = jnp.max(jnp.where(win, pz, ninf), axis=(0, 1),
                                      keepdims=True)
        return 0

    jax.lax.fori_loop(1, SAMPLE, step, 0)


def _fps_pallas(posb):
    p = jnp.transpose(posb, (0, 2, 1))
    p = jnp.pad(p, ((0, 0), (0, 0), (0, NPAD - N_PER)))
    p = p.reshape(B, 3, ROWS, 128)
    out = pl.pallas_call(
        _fps_body,
        out_shape=jax.ShapeDtypeStruct((B, OROWS, 128), jnp.int32),
        scratch_shapes=[
            pltpu.VMEM((B, ROWS, 128), jnp.float32),
            pltpu.VMEM((B, 3, 1, 1), jnp.float32),
        ],
    )(p)
    return out.reshape(B, SAMPLE)


def kernel(pos, edge_index, params):
    src = edge_index[0]
    dst = edge_index[1]
    deg = jnp.clip(jnp.zeros((N,), jnp.float32).at[dst].add(1.0), 1.0, None)
    feats = [pos]
    f = pos
    for r in range(R):
        f = jnp.zeros((N, 3), jnp.float32).at[dst].add(f[src]) / deg[:, None]
        feats.append(f)

    hs = [_ffn(feats[r], params['inc%d_W0' % r], params['inc%d_b0' % r],
               params['inc%d_W1' % r], params['inc%d_b1' % r])
          for r in range(R + 1)]
    x_ = _ffn(jax.nn.relu(jnp.concatenate(hs, axis=1)),
              params['proj_W0'], params['proj_b0'],
              params['proj_W1'], params['proj_b1'])
    xb = x_.reshape(B, N_PER, 64)
    gate = jax.nn.softmax(xb @ params['gate_W'] + params['gate_b'], axis=1)
    gfeat = jnp.broadcast_to(
        jnp.sum(gate * xb, axis=1, keepdims=True), (B, N_PER, 64))
    p = _bn(jax.nn.relu(pos @ params['pm_W0'] + params['pm_b0']),
            params['bn0_g'], params['bn0_b'])
    p = _bn(jax.nn.relu(p @ params['pm_W1'] + params['pm_b1']),
            params['bn1_g'], params['bn1_b'])
    p = (p @ params['pm_W2'] + params['pm_b2']).reshape(B, N_PER, 64)
    x = jnp.concatenate([xb, p, gfeat], axis=2)

    posb = pos.reshape(B, N_PER, 3)
    idxs = _fps_pallas(posb)

    sp = jnp.take_along_axis(posb, idxs[:, :, None], axis=1)
    sx = jnp.take_along_axis(x, idxs[:, :, None], axis=1)
    d2 = jnp.sum((sp[:, :, None, :] - sp[:, None, :, :]) ** 2, axis=-1)
    _, nbr = jax.lax.top_k(-d2, KNN)

    t = sx @ params['theta_W']
    pphi = sx @ params['phi_W']
    t_nb = jax.vmap(lambda ts, nb: ts[nb])(t, nbr)
    tmax = jnp.max(t_nb, axis=2)
    x_out = (tmax - t + pphi + params['theta_b'] + params['phi_b']
             ).reshape(B * SAMPLE, 256)

    offs = (jnp.arange(B) * SAMPLE)[:, None]
    new_src = (nbr.reshape(B, -1) + offs).reshape(-1)
    new_dst = (jnp.tile(jnp.repeat(jnp.arange(SAMPLE), KNN)[None, :],
                        (B, 1)) + offs).reshape(-1)
    return (x_out, jnp.stack([new_src, new_dst]))


# speedup vs baseline: 1.6751x; 1.0457x over previous
import jax
import jax.numpy as jnp
import numpy as np
from jax.experimental import pallas as pl
from jax.experimental.pallas import tpu as pltpu

B = 2
N_PER = 5000
N = B * N_PER
R = 4
SAMPLE = 2048
KNN = 32


def _ffn(x, W0, b0, W1, b1):
    return jax.nn.relu(x @ W0 + b0) @ W1 + b1


def _bn(x, g, b):
    mu = jnp.mean(x, axis=0)
    var = jnp.var(x, axis=0)
    return (x - mu) / jnp.sqrt(var + 1e-5) * g + b


ROWS = 40
NPAD = ROWS * 128
OROWS = SAMPLE // 128
BIG_I = np.int32(2**30)


def _fps_body(pos_ref, out_ref, sp_ref, dists_ref, coord_ref):
    flat = (jax.lax.broadcasted_iota(jnp.int32, (ROWS, 128), 0) * 128
            + jax.lax.broadcasted_iota(jnp.int32, (ROWS, 128), 1))
    oflat = (jax.lax.broadcasted_iota(jnp.int32, (OROWS, 128), 0) * 128
             + jax.lax.broadcasted_iota(jnp.int32, (OROWS, 128), 1))
    valid = flat < N_PER
    ninf = jnp.float32(-jnp.inf)
    for b in range(B):
        dists_ref[b] = jnp.where(valid, jnp.float32(1e10), ninf)
        out_ref[b] = jnp.zeros((OROWS, 128), jnp.int32)
        for c in range(3):
            coord_ref[b, c] = pos_ref[b, c, 0:1, 0:1]
            sp_ref[b, c] = jnp.where(
                oflat == 0, pos_ref[b, c, 0:1, 0:1], jnp.float32(0.0))

    def step(k, _):
        for b in range(B):
            px = pos_ref[b, 0]
            py = pos_ref[b, 1]
            pz = pos_ref[b, 2]
            dx = px - coord_ref[b, 0]
            dy = py - coord_ref[b, 1]
            dz = pz - coord_ref[b, 2]
            d = (dx * dx + dy * dy) + dz * dz
            nd = jnp.minimum(dists_ref[b], d)
            dists_ref[b] = nd
            m = jnp.max(nd, axis=(0, 1), keepdims=True)
            idx = jnp.min(jnp.where(nd == m, flat, BIG_I), axis=(0, 1),
                          keepdims=True)
            win = flat == idx
            out_ref[b] = jnp.where(oflat == k, idx[0, 0], out_ref[b])
            for c, pc in ((0, px), (1, py), (2, pz)):
                cv = jnp.max(jnp.where(win, pc, ninf), axis=(0, 1),
                             keepdims=True)
                coord_ref[b, c] = cv
                sp_ref[b, c] = jnp.where(oflat == k, cv, sp_ref[b, c])
        return 0

    jax.lax.fori_loop(1, SAMPLE, step, 0)


def _fps_pallas(posb):
    p = jnp.transpose(posb, (0, 2, 1))
    p = jnp.pad(p, ((0, 0), (0, 0), (0, NPAD - N_PER)))
    p = p.reshape(B, 3, ROWS, 128)
    out, sp = pl.pallas_call(
        _fps_body,
        out_shape=(jax.ShapeDtypeStruct((B, OROWS, 128), jnp.int32),
                   jax.ShapeDtypeStruct((B, 3, OROWS, 128), jnp.float32)),
        scratch_shapes=[
            pltpu.VMEM((B, ROWS, 128), jnp.float32),
            pltpu.VMEM((B, 3, 1, 1), jnp.float32),
        ],
    )(p)
    return out.reshape(B, SAMPLE), sp.reshape(B, 3, SAMPLE)


KTILE = 256
BIG_F = np.float32(np.inf)


def _knn_body(spt_ref, spr_ref, nbr_ref):
    iota_j = jax.lax.broadcasted_iota(jnp.int32, (KTILE, SAMPLE), 1)
    dx = spt_ref[0, :, 0:1] - spr_ref[0, 0:1, :]
    dy = spt_ref[0, :, 1:2] - spr_ref[0, 1:2, :]
    dz = spt_ref[0, :, 2:3] - spr_ref[0, 2:3, :]
    d2 = (dx * dx + dy * dy) + dz * dz
    for k in range(KNN):
        m = jnp.min(d2, axis=1, keepdims=True)
        idx = jnp.min(jnp.where(d2 == m, iota_j, BIG_I), axis=1,
                      keepdims=True)
        nbr_ref[0, :, k:k + 1] = idx
        d2 = jnp.where(iota_j == idx, BIG_F, d2)


def _knn_pallas(sp):
    spt = jnp.transpose(sp, (0, 2, 1))
    grid = (B, SAMPLE // KTILE)
    return pl.pallas_call(
        _knn_body,
        grid=grid,
        in_specs=[
            pl.BlockSpec((1, KTILE, 3), lambda b, i: (b, i, 0)),
            pl.BlockSpec((1, 3, SAMPLE), lambda b, i: (b, 0, 0)),
        ],
        out_specs=pl.BlockSpec((1, KTILE, KNN), lambda b, i: (b, i, 0)),
        out_shape=jax.ShapeDtypeStruct((B, SAMPLE, KNN), jnp.int32),
    )(spt, sp)


def kernel(pos, edge_index, params):
    src = edge_index[0]
    dst = edge_index[1]
    deg = jnp.clip(jnp.zeros((N,), jnp.float32).at[dst].add(1.0), 1.0, None)
    feats = [pos]
    f = pos
    for r in range(R):
        f = jnp.zeros((N, 3), jnp.float32).at[dst].add(f[src]) / deg[:, None]
        feats.append(f)

    hs = [_ffn(feats[r], params['inc%d_W0' % r], params['inc%d_b0' % r],
               params['inc%d_W1' % r], params['inc%d_b1' % r])
          for r in range(R + 1)]
    x_ = _ffn(jax.nn.relu(jnp.concatenate(hs, axis=1)),
              params['proj_W0'], params['proj_b0'],
              params['proj_W1'], params['proj_b1'])
    xb = x_.reshape(B, N_PER, 64)
    gate = jax.nn.softmax(xb @ params['gate_W'] + params['gate_b'], axis=1)
    gfeat = jnp.broadcast_to(
        jnp.sum(gate * xb, axis=1, keepdims=True), (B, N_PER, 64))
    p = _bn(jax.nn.relu(pos @ params['pm_W0'] + params['pm_b0']),
            params['bn0_g'], params['bn0_b'])
    p = _bn(jax.nn.relu(p @ params['pm_W1'] + params['pm_b1']),
            params['bn1_g'], params['bn1_b'])
    p = (p @ params['pm_W2'] + params['pm_b2']).reshape(B, N_PER, 64)
    x = jnp.concatenate([xb, p, gfeat], axis=2)

    posb = pos.reshape(B, N_PER, 3)
    idxs, sp3 = _fps_pallas(posb)

    sx = jnp.take_along_axis(x, idxs[:, :, None], axis=1)
    nbr = _knn_pallas(sp3)

    t = sx @ params['theta_W']
    pphi = sx @ params['phi_W']
    t_nb = jax.vmap(lambda ts, nb: ts[nb])(t, nbr)
    tmax = jnp.max(t_nb, axis=2)
    x_out = (tmax - t + pphi + params['theta_b'] + params['phi_b']
             ).reshape(B * SAMPLE, 256)

    offs = (jnp.arange(B) * SAMPLE)[:, None]
    new_src = (nbr.reshape(B, -1) + offs).reshape(-1)
    new_dst = (jnp.tile(jnp.repeat(jnp.arange(SAMPLE), KNN)[None, :],
                        (B, 1)) + offs).reshape(-1)
    return (x_out, jnp.stack([new_src, new_dst]))


# speedup vs baseline: 3.7936x; 2.2647x over previous
import functools

import jax
import jax.numpy as jnp
import numpy as np
from jax import lax
from jax.experimental import pallas as pl
from jax.experimental.pallas import tpu as pltpu
from jax.experimental.pallas import tpu_sc as plsc

B = 2
N_PER = 5000
N = B * N_PER
R = 4
SAMPLE = 2048
KNN = 32

E_PER = 160000
NS = 16
EPT = E_PER // NS
NLOC = 6144
SLICE = NLOC // NS


def _zero_plane(ref, n):
    z = jnp.zeros((16,), jnp.float32)

    def body(i, _):
        ref[pl.ds(i * 16, 16)] = z
        return 0
    lax.fori_loop(0, n // 16, body, 0, unroll=8)


def _tile_reduce(c, s, bufs, shpart, shtot, rdbuf, sumbuf, nplanes):
    for cp in range(nplanes):
        pltpu.sync_copy(bufs[cp], shpart.at[s, cp, 0])
    plsc.subcore_barrier()
    base = s * SLICE
    for cp in range(nplanes):
        for j in range(NS):
            pltpu.sync_copy(shpart.at[j, cp, 0, pl.ds(base, SLICE)], rdbuf.at[j, 0])

        def sbody(g, _):
            sl = pl.ds(g * 16, 16)
            v = rdbuf[0, 0, sl]
            for j in range(1, NS):
                v = v + rdbuf[j, 0, sl]
            sumbuf[sl] = v
            return 0
        lax.fori_loop(0, SLICE // 16, sbody, 0, unroll=4)
        pltpu.sync_copy(sumbuf, shtot.at[cp, 0, pl.ds(base, SLICE)])
    plsc.subcore_barrier()


def _sign_body(posT, srcl, dstl, out, fx, fy, fz, ax, ay, az, dg, sidx, didx,
               rdbuf, sumbuf, shpart, shtot):
    c = lax.axis_index("c")
    s = lax.axis_index("s")
    pltpu.sync_copy(srcl.at[c, s, 0], sidx)
    pltpu.sync_copy(dstl.at[c, s, 0], didx)
    pltpu.sync_copy(posT.at[c, 0, 0], fx)
    pltpu.sync_copy(posT.at[c, 1, 0], fy)
    pltpu.sync_copy(posT.at[c, 2, 0], fz)

    _zero_plane(ax, NLOC)
    ones = jnp.ones((16,), jnp.float32)

    def degbody(i, _):
        d16 = didx[pl.ds(i * 16, 16)]
        plsc.addupdate_scatter(ax, [d16], ones)
        return 0
    lax.fori_loop(0, EPT // 16, degbody, 0)

    _tile_reduce(c, s, [ax], shpart, shtot, rdbuf, sumbuf, 1)
    pltpu.sync_copy(shtot.at[0, 0], dg)

    def degclip(i, _):
        sl = pl.ds(i * 16, 16)
        dg[sl] = jnp.maximum(dg[sl], 1.0)
        return 0
    lax.fori_loop(0, NLOC // 16, degclip, 0, unroll=8)

    for r in range(R):
        _zero_plane(ax, NLOC)
        _zero_plane(ay, NLOC)
        _zero_plane(az, NLOC)

        def ebody(i, _):
            s16 = sidx[pl.ds(i * 16, 16)]
            d16 = didx[pl.ds(i * 16, 16)]
            plsc.addupdate_scatter(ax, [d16], plsc.load_gather(fx, [s16]))
            plsc.addupdate_scatter(ay, [d16], plsc.load_gather(fy, [s16]))
            plsc.addupdate_scatter(az, [d16], plsc.load_gather(fz, [s16]))
            return 0
        lax.fori_loop(0, EPT // 16, ebody, 0)

        _tile_reduce(c, s, [ax, ay, az], shpart, shtot, rdbuf, sumbuf, 3)
        pltpu.sync_copy(shtot.at[0, 0], fx)
        pltpu.sync_copy(shtot.at[1, 0], fy)
        pltpu.sync_copy(shtot.at[2, 0], fz)

        def divbody(i, _):
            sl = pl.ds(i * 16, 16)
            d16 = dg[sl]
            fx[sl] = fx[sl] / d16
            fy[sl] = fy[sl] / d16
            fz[sl] = fz[sl] / d16
            return 0
        lax.fori_loop(0, NLOC // 16, divbody, 0, unroll=8)

        @pl.when(s == 0)
        def _():
            pltpu.sync_copy(fx, out.at[r, c, 0, 0])
            pltpu.sync_copy(fy, out.at[r, c, 1, 0])
            pltpu.sync_copy(fz, out.at[r, c, 2, 0])
        plsc.subcore_barrier()


def _sign_sc(pos, edge_index):
    posT = jnp.transpose(pos.reshape(B, N_PER, 3), (0, 2, 1))
    posT = jnp.pad(posT, ((0, 0), (0, 0), (0, NLOC - N_PER)))
    posT = posT.reshape(B, 3, 1, NLOC)
    offs = (jnp.arange(B, dtype=jnp.int32) * N_PER)[:, None, None]
    srcl = edge_index[0].reshape(B, NS, 1, EPT).astype(jnp.int32) - offs[:, :, None]
    dstl = edge_index[1].reshape(B, NS, 1, EPT).astype(jnp.int32) - offs[:, :, None]
    mesh = plsc.VectorSubcoreMesh(core_axis_name="c", subcore_axis_name="s")
    kfn = functools.partial(
        pl.kernel,
        mesh=mesh,
        compiler_params=pltpu.CompilerParams(needs_layout_passes=False),
        out_type=jax.ShapeDtypeStruct((R, B, 3, 1, NLOC), jnp.float32),
        scratch_types=[
            pltpu.VMEM((NLOC,), jnp.float32),
            pltpu.VMEM((NLOC,), jnp.float32),
            pltpu.VMEM((NLOC,), jnp.float32),
            pltpu.VMEM((NLOC,), jnp.float32),
            pltpu.VMEM((NLOC,), jnp.float32),
            pltpu.VMEM((NLOC,), jnp.float32),
            pltpu.VMEM((NLOC,), jnp.float32),
            pltpu.VMEM((EPT,), jnp.int32),
            pltpu.VMEM((EPT,), jnp.int32),
            pltpu.VMEM((NS, 1, SLICE), jnp.float32),
            pltpu.VMEM((SLICE,), jnp.float32),
            pltpu.VMEM_SHARED((NS, 3, 1, NLOC), jnp.float32),
            pltpu.VMEM_SHARED((3, 1, NLOC), jnp.float32),
        ],
    )(_sign_body)
    feats = kfn(posT, srcl, dstl)
    feats = feats[:, :, :, 0, :N_PER]
    return jnp.transpose(feats, (0, 1, 3, 2)).reshape(R, B * N_PER, 3)


def _ffn(x, W0, b0, W1, b1):
    return jax.nn.relu(x @ W0 + b0) @ W1 + b1


def _bn(x, g, b):
    mu = jnp.mean(x, axis=0)
    var = jnp.var(x, axis=0)
    return (x - mu) / jnp.sqrt(var + 1e-5) * g + b


ROWS = 40
NPAD = ROWS * 128
OROWS = SAMPLE // 128
BIG_I = np.int32(2**30)


def _fps_body(pos_ref, out_ref, sp_ref, dists_ref, coord_ref):
    flat = (jax.lax.broadcasted_iota(jnp.int32, (ROWS, 128), 0) * 128
            + jax.lax.broadcasted_iota(jnp.int32, (ROWS, 128), 1))
    oflat = (jax.lax.broadcasted_iota(jnp.int32, (OROWS, 128), 0) * 128
             + jax.lax.broadcasted_iota(jnp.int32, (OROWS, 128), 1))
    valid = flat < N_PER
    ninf = jnp.float32(-jnp.inf)
    for b in range(B):
        dists_ref[b] = jnp.where(valid, jnp.float32(1e10), ninf)
        out_ref[b] = jnp.zeros((OROWS, 128), jnp.int32)
        for c in range(3):
            coord_ref[b, c] = pos_ref[b, c, 0:1, 0:1]
            sp_ref[b, c] = jnp.where(
                oflat == 0, pos_ref[b, c, 0:1, 0:1], jnp.float32(0.0))

    def step(k, _):
        for b in range(B):
            px = pos_ref[b, 0]
            py = pos_ref[b, 1]
            pz = pos_ref[b, 2]
            dx = px - coord_ref[b, 0]
            dy = py - coord_ref[b, 1]
            dz = pz - coord_ref[b, 2]
            d = (dx * dx + dy * dy) + dz * dz
            nd = jnp.minimum(dists_ref[b], d)
            dists_ref[b] = nd
            m = jnp.max(nd, axis=(0, 1), keepdims=True)
            idx = jnp.min(jnp.where(nd == m, flat, BIG_I), axis=(0, 1),
                          keepdims=True)
            win = flat == idx
            out_ref[b] = jnp.where(oflat == k, idx[0, 0], out_ref[b])
            for c, pc in ((0, px), (1, py), (2, pz)):
                cv = jnp.max(jnp.where(win, pc, ninf), axis=(0, 1),
                             keepdims=True)
                coord_ref[b, c] = cv
                sp_ref[b, c] = jnp.where(oflat == k, cv, sp_ref[b, c])
        return 0

    jax.lax.fori_loop(1, SAMPLE, step, 0)


def _fps_pallas(posb):
    p = jnp.transpose(posb, (0, 2, 1))
    p = jnp.pad(p, ((0, 0), (0, 0), (0, NPAD - N_PER)))
    p = p.reshape(B, 3, ROWS, 128)
    out, sp = pl.pallas_call(
        _fps_body,
        out_shape=(jax.ShapeDtypeStruct((B, OROWS, 128), jnp.int32),
                   jax.ShapeDtypeStruct((B, 3, OROWS, 128), jnp.float32)),
        scratch_shapes=[
            pltpu.VMEM((B, ROWS, 128), jnp.float32),
            pltpu.VMEM((B, 3, 1, 1), jnp.float32),
        ],
    )(p)
    return out.reshape(B, SAMPLE), sp.reshape(B, 3, SAMPLE)


KTILE = 256
BIG_F = np.float32(np.inf)


def _knn_body(spt_ref, spr_ref, nbr_ref):
    iota_j = jax.lax.broadcasted_iota(jnp.int32, (KTILE, SAMPLE), 1)
    dx = spt_ref[0, :, 0:1] - spr_ref[0, 0:1, :]
    dy = spt_ref[0, :, 1:2] - spr_ref[0, 1:2, :]
    dz = spt_ref[0, :, 2:3] - spr_ref[0, 2:3, :]
    d2 = (dx * dx + dy * dy) + dz * dz
    for k in range(KNN):
        m = jnp.min(d2, axis=1, keepdims=True)
        idx = jnp.min(jnp.where(d2 == m, iota_j, BIG_I), axis=1,
                      keepdims=True)
        nbr_ref[0, :, k:k + 1] = idx
        d2 = jnp.where(iota_j == idx, BIG_F, d2)


def _knn_pallas(sp):
    spt = jnp.transpose(sp, (0, 2, 1))
    grid = (B, SAMPLE // KTILE)
    return pl.pallas_call(
        _knn_body,
        grid=grid,
        in_specs=[
            pl.BlockSpec((1, KTILE, 3), lambda b, i: (b, i, 0)),
            pl.BlockSpec((1, 3, SAMPLE), lambda b, i: (b, 0, 0)),
        ],
        out_specs=pl.BlockSpec((1, KTILE, KNN), lambda b, i: (b, i, 0)),
        out_shape=jax.ShapeDtypeStruct((B, SAMPLE, KNN), jnp.int32),
    )(spt, sp)


def kernel(pos, edge_index, params):
    hop = _sign_sc(pos, edge_index)
    feats = [pos] + [hop[r] for r in range(R)]

    hs = [_ffn(feats[r], params['inc%d_W0' % r], params['inc%d_b0' % r],
               params['inc%d_W1' % r], params['inc%d_b1' % r])
          for r in range(R + 1)]
    x_ = _ffn(jax.nn.relu(jnp.concatenate(hs, axis=1)),
              params['proj_W0'], params['proj_b0'],
              params['proj_W1'], params['proj_b1'])
    xb = x_.reshape(B, N_PER, 64)
    gate = jax.nn.softmax(xb @ params['gate_W'] + params['gate_b'], axis=1)
    gfeat = jnp.broadcast_to(
        jnp.sum(gate * xb, axis=1, keepdims=True), (B, N_PER, 64))
    p = _bn(jax.nn.relu(pos @ params['pm_W0'] + params['pm_b0']),
            params['bn0_g'], params['bn0_b'])
    p = _bn(jax.nn.relu(p @ params['pm_W1'] + params['pm_b1']),
            params['bn1_g'], params['bn1_b'])
    p = (p @ params['pm_W2'] + params['pm_b2']).reshape(B, N_PER, 64)
    x = jnp.concatenate([xb, p, gfeat], axis=2)

    posb = pos.reshape(B, N_PER, 3)
    idxs, sp3 = _fps_pallas(posb)

    sx = jnp.take_along_axis(x, idxs[:, :, None], axis=1)
    nbr = _knn_pallas(sp3)

    t = sx @ params['theta_W']
    pphi = sx @ params['phi_W']
    t_nb = jax.vmap(lambda ts, nb: ts[nb])(t, nbr)
    tmax = jnp.max(t_nb, axis=2)
    x_out = (tmax - t + pphi + params['theta_b'] + params['phi_b']
             ).reshape(B * SAMPLE, 256)

    offs = (jnp.arange(B) * SAMPLE)[:, None]
    new_src = (nbr.reshape(B, -1) + offs).reshape(-1)
    new_dst = (jnp.tile(jnp.repeat(jnp.arange(SAMPLE), KNN)[None, :],
                        (B, 1)) + offs).reshape(-1)
    return (x_out, jnp.stack([new_src, new_dst]))


# speedup vs baseline: 6.4648x; 1.7041x over previous
import functools

import jax
import jax.numpy as jnp
import numpy as np
from jax import lax
from jax.experimental import pallas as pl
from jax.experimental.pallas import tpu as pltpu
from jax.experimental.pallas import tpu_sc as plsc

B = 2
N_PER = 5000
N = B * N_PER
R = 4
SAMPLE = 2048
KNN = 32

E_PER = 160000
NS = 16
EPT = E_PER // NS
NLOC = 6144
SLICE = NLOC // NS


def _zero_plane(ref, n):
    z = jnp.zeros((16,), jnp.float32)

    def body(i, _):
        ref[pl.ds(i * 16, 16)] = z
        return 0
    lax.fori_loop(0, n // 16, body, 0, unroll=8)


def _tile_reduce(c, s, bufs, shpart, shtot, rdbuf, sumbuf, nplanes):
    for cp in range(nplanes):
        pltpu.sync_copy(bufs[cp], shpart.at[s, cp, 0])
    plsc.subcore_barrier()
    base = s * SLICE
    for cp in range(nplanes):
        for j in range(NS):
            pltpu.sync_copy(shpart.at[j, cp, 0, pl.ds(base, SLICE)], rdbuf.at[j, 0])

        def sbody(g, _):
            sl = pl.ds(g * 16, 16)
            v = rdbuf[0, 0, sl]
            for j in range(1, NS):
                v = v + rdbuf[j, 0, sl]
            sumbuf[sl] = v
            return 0
        lax.fori_loop(0, SLICE // 16, sbody, 0, unroll=4)
        pltpu.sync_copy(sumbuf, shtot.at[cp, 0, pl.ds(base, SLICE)])
    plsc.subcore_barrier()


def _sign_body(posT, srcl, dstl, out, fx, fy, fz, ax, ay, az, dg, sidx, didx,
               rdbuf, sumbuf, shpart, shtot):
    c = lax.axis_index("c")
    s = lax.axis_index("s")
    pltpu.sync_copy(srcl.at[c, s, 0], sidx)
    pltpu.sync_copy(dstl.at[c, s, 0], didx)
    pltpu.sync_copy(posT.at[c, 0, 0], fx)
    pltpu.sync_copy(posT.at[c, 1, 0], fy)
    pltpu.sync_copy(posT.at[c, 2, 0], fz)

    _zero_plane(ax, NLOC)
    ones = jnp.ones((16,), jnp.float32)

    def degbody(i, _):
        d16 = didx[pl.ds(i * 16, 16)]
        plsc.addupdate_scatter(ax, [d16], ones)
        return 0
    lax.fori_loop(0, EPT // 16, degbody, 0)

    _tile_reduce(c, s, [ax], shpart, shtot, rdbuf, sumbuf, 1)
    pltpu.sync_copy(shtot.at[0, 0], dg)

    def degclip(i, _):
        sl = pl.ds(i * 16, 16)
        dg[sl] = jnp.maximum(dg[sl], 1.0)
        return 0
    lax.fori_loop(0, NLOC // 16, degclip, 0, unroll=8)

    for r in range(R):
        _zero_plane(ax, NLOC)
        _zero_plane(ay, NLOC)
        _zero_plane(az, NLOC)

        def ebody(i, _):
            s16 = sidx[pl.ds(i * 16, 16)]
            d16 = didx[pl.ds(i * 16, 16)]
            plsc.addupdate_scatter(ax, [d16], plsc.load_gather(fx, [s16]))
            plsc.addupdate_scatter(ay, [d16], plsc.load_gather(fy, [s16]))
            plsc.addupdate_scatter(az, [d16], plsc.load_gather(fz, [s16]))
            return 0
        lax.fori_loop(0, EPT // 16, ebody, 0)

        _tile_reduce(c, s, [ax, ay, az], shpart, shtot, rdbuf, sumbuf, 3)
        pltpu.sync_copy(shtot.at[0, 0], fx)
        pltpu.sync_copy(shtot.at[1, 0], fy)
        pltpu.sync_copy(shtot.at[2, 0], fz)

        def divbody(i, _):
            sl = pl.ds(i * 16, 16)
            d16 = dg[sl]
            fx[sl] = fx[sl] / d16
            fy[sl] = fy[sl] / d16
            fz[sl] = fz[sl] / d16
            return 0
        lax.fori_loop(0, NLOC // 16, divbody, 0, unroll=8)

        @pl.when(s == 0)
        def _():
            pltpu.sync_copy(fx, out.at[r, c, 0, 0])
            pltpu.sync_copy(fy, out.at[r, c, 1, 0])
            pltpu.sync_copy(fz, out.at[r, c, 2, 0])
        plsc.subcore_barrier()


def _sign_sc(pos, edge_index):
    posT = jnp.transpose(pos.reshape(B, N_PER, 3), (0, 2, 1))
    posT = jnp.pad(posT, ((0, 0), (0, 0), (0, NLOC - N_PER)))
    posT = posT.reshape(B, 3, 1, NLOC)
    offs = (jnp.arange(B, dtype=jnp.int32) * N_PER)[:, None, None]
    srcl = edge_index[0].reshape(B, NS, 1, EPT).astype(jnp.int32) - offs[:, :, None]
    dstl = edge_index[1].reshape(B, NS, 1, EPT).astype(jnp.int32) - offs[:, :, None]
    mesh = plsc.VectorSubcoreMesh(core_axis_name="c", subcore_axis_name="s")
    kfn = functools.partial(
        pl.kernel,
        mesh=mesh,
        compiler_params=pltpu.CompilerParams(needs_layout_passes=False),
        out_type=jax.ShapeDtypeStruct((R, B, 3, 1, NLOC), jnp.float32),
        scratch_types=[
            pltpu.VMEM((NLOC,), jnp.float32),
            pltpu.VMEM((NLOC,), jnp.float32),
            pltpu.VMEM((NLOC,), jnp.float32),
            pltpu.VMEM((NLOC,), jnp.float32),
            pltpu.VMEM((NLOC,), jnp.float32),
            pltpu.VMEM((NLOC,), jnp.float32),
            pltpu.VMEM((NLOC,), jnp.float32),
            pltpu.VMEM((EPT,), jnp.int32),
            pltpu.VMEM((EPT,), jnp.int32),
            pltpu.VMEM((NS, 1, SLICE), jnp.float32),
            pltpu.VMEM((SLICE,), jnp.float32),
            pltpu.VMEM_SHARED((NS, 3, 1, NLOC), jnp.float32),
            pltpu.VMEM_SHARED((3, 1, NLOC), jnp.float32),
        ],
    )(_sign_body)
    feats = kfn(posT, srcl, dstl)
    feats = feats[:, :, :, 0, :N_PER]
    return jnp.transpose(feats, (0, 1, 3, 2)).reshape(R, B * N_PER, 3)


NOUT = B * SAMPLE
DD = 256
NW = 32
RPW = NOUT // NW
CHG = 8
NCHG = RPW // CHG


def _gmax_body(t_hbm, gn_hbm, out_hbm, idxbuf, rows_v, chunkout, sem):
    c = lax.axis_index("c")
    s = lax.axis_index("s")
    wid = s * 2 + c
    base = wid * RPW * KNN
    pltpu.sync_copy(gn_hbm.at[pl.ds(base, RPW * KNN)], idxbuf)

    def chunk(ci, _):
        pltpu.async_copy(
            t_hbm.at[idxbuf.at[pl.ds(ci * (CHG * KNN), CHG * KNN)]],
            rows_v, sem).wait()

        def lanes(lc, _):
            sl = pl.ds(lc * 16, 16)
            for r in range(CHG):
                m = rows_v[r * KNN, sl]
                for j in range(1, KNN):
                    m = jnp.maximum(m, rows_v[r * KNN + j, sl])
                chunkout[r, sl] = m
            return 0
        lax.fori_loop(0, DD // 16, lanes, 0)
        pltpu.sync_copy(chunkout,
                        out_hbm.at[pl.ds(wid * RPW + ci * CHG, CHG)])
        return 0
    lax.fori_loop(0, NCHG, chunk, 0)


def _gather_max_sc(t, gn):
    mesh = plsc.VectorSubcoreMesh(core_axis_name="c", subcore_axis_name="s")
    kfn = functools.partial(
        pl.kernel,
        mesh=mesh,
        compiler_params=pltpu.CompilerParams(needs_layout_passes=False),
        out_type=jax.ShapeDtypeStruct((NOUT, DD), jnp.float32),
        scratch_types=[
            pltpu.VMEM((RPW * KNN,), jnp.int32),
            pltpu.VMEM((CHG * KNN, DD), jnp.float32),
            pltpu.VMEM((CHG, DD), jnp.float32),
            pltpu.SemaphoreType.DMA,
        ],
    )(_gmax_body)
    return kfn(t, gn)


def _ffn(x, W0, b0, W1, b1):
    return jax.nn.relu(x @ W0 + b0) @ W1 + b1


def _bn(x, g, b):
    mu = jnp.mean(x, axis=0)
    var = jnp.var(x, axis=0)
    return (x - mu) / jnp.sqrt(var + 1e-5) * g + b


ROWS = 40
NPAD = ROWS * 128
OROWS = SAMPLE // 128
BIG_I = np.int32(2**30)


def _fps_body(pos_ref, out_ref, sp_ref, dists_ref, coord_ref):
    flat = (jax.lax.broadcasted_iota(jnp.int32, (ROWS, 128), 0) * 128
            + jax.lax.broadcasted_iota(jnp.int32, (ROWS, 128), 1))
    oflat = (jax.lax.broadcasted_iota(jnp.int32, (OROWS, 128), 0) * 128
             + jax.lax.broadcasted_iota(jnp.int32, (OROWS, 128), 1))
    valid = flat < N_PER
    ninf = jnp.float32(-jnp.inf)
    for b in range(B):
        dists_ref[b] = jnp.where(valid, jnp.float32(1e10), ninf)
        out_ref[b] = jnp.zeros((OROWS, 128), jnp.int32)
        for c in range(3):
            coord_ref[b, c] = pos_ref[b, c, 0:1, 0:1]
            sp_ref[b, c] = jnp.where(
                oflat == 0, pos_ref[b, c, 0:1, 0:1], jnp.float32(0.0))

    def step(k, _):
        for b in range(B):
            px = pos_ref[b, 0]
            py = pos_ref[b, 1]
            pz = pos_ref[b, 2]
            dx = px - coord_ref[b, 0]
            dy = py - coord_ref[b, 1]
            dz = pz - coord_ref[b, 2]
            d = (dx * dx + dy * dy) + dz * dz
            nd = jnp.minimum(dists_ref[b], d)
            dists_ref[b] = nd
            m = jnp.max(nd, axis=(0, 1), keepdims=True)
            idx = jnp.min(jnp.where(nd == m, flat, BIG_I), axis=(0, 1),
                          keepdims=True)
            win = flat == idx
            out_ref[b] = jnp.where(oflat == k, idx[0, 0], out_ref[b])
            for c, pc in ((0, px), (1, py), (2, pz)):
                cv = jnp.max(jnp.where(win, pc, ninf), axis=(0, 1),
                             keepdims=True)
                coord_ref[b, c] = cv
                sp_ref[b, c] = jnp.where(oflat == k, cv, sp_ref[b, c])
        return 0

    jax.lax.fori_loop(1, SAMPLE, step, 0)


def _fps_pallas(posb):
    p = jnp.transpose(posb, (0, 2, 1))
    p = jnp.pad(p, ((0, 0), (0, 0), (0, NPAD - N_PER)))
    p = p.reshape(B, 3, ROWS, 128)
    out, sp = pl.pallas_call(
        _fps_body,
        out_shape=(jax.ShapeDtypeStruct((B, OROWS, 128), jnp.int32),
                   jax.ShapeDtypeStruct((B, 3, OROWS, 128), jnp.float32)),
        scratch_shapes=[
            pltpu.VMEM((B, ROWS, 128), jnp.float32),
            pltpu.VMEM((B, 3, 1, 1), jnp.float32),
        ],
    )(p)
    return out.reshape(B, SAMPLE), sp.reshape(B, 3, SAMPLE)


KTILE = 256
BIG_F = np.float32(np.inf)


def _knn_body(spt_ref, spr_ref, nbr_ref):
    iota_j = jax.lax.broadcasted_iota(jnp.int32, (KTILE, SAMPLE), 1)
    dx = spt_ref[0, :, 0:1] - spr_ref[0, 0:1, :]
    dy = spt_ref[0, :, 1:2] - spr_ref[0, 1:2, :]
    dz = spt_ref[0, :, 2:3] - spr_ref[0, 2:3, :]
    d2 = (dx * dx + dy * dy) + dz * dz
    for k in range(KNN):
        m = jnp.min(d2, axis=1, keepdims=True)
        idx = jnp.min(jnp.where(d2 == m, iota_j, BIG_I), axis=1,
                      keepdims=True)
        nbr_ref[0, :, k:k + 1] = idx
        d2 = jnp.where(iota_j == idx, BIG_F, d2)


def _knn_pallas(sp):
    spt = jnp.transpose(sp, (0, 2, 1))
    grid = (B, SAMPLE // KTILE)
    return pl.pallas_call(
        _knn_body,
        grid=grid,
        in_specs=[
            pl.BlockSpec((1, KTILE, 3), lambda b, i: (b, i, 0)),
            pl.BlockSpec((1, 3, SAMPLE), lambda b, i: (b, 0, 0)),
        ],
        out_specs=pl.BlockSpec((1, KTILE, KNN), lambda b, i: (b, i, 0)),
        out_shape=jax.ShapeDtypeStruct((B, SAMPLE, KNN), jnp.int32),
    )(spt, sp)


def kernel(pos, edge_index, params):
    hop = _sign_sc(pos, edge_index)
    feats = [pos] + [hop[r] for r in range(R)]

    hs = [_ffn(feats[r], params['inc%d_W0' % r], params['inc%d_b0' % r],
               params['inc%d_W1' % r], params['inc%d_b1' % r])
          for r in range(R + 1)]
    x_ = _ffn(jax.nn.relu(jnp.concatenate(hs, axis=1)),
              params['proj_W0'], params['proj_b0'],
              params['proj_W1'], params['proj_b1'])
    xb = x_.reshape(B, N_PER, 64)
    gate = jax.nn.softmax(xb @ params['gate_W'] + params['gate_b'], axis=1)
    gfeat = jnp.broadcast_to(
        jnp.sum(gate * xb, axis=1, keepdims=True), (B, N_PER, 64))
    p = _bn(jax.nn.relu(pos @ params['pm_W0'] + params['pm_b0']),
            params['bn0_g'], params['bn0_b'])
    p = _bn(jax.nn.relu(p @ params['pm_W1'] + params['pm_b1']),
            params['bn1_g'], params['bn1_b'])
    p = (p @ params['pm_W2'] + params['pm_b2']).reshape(B, N_PER, 64)
    x = jnp.concatenate([xb, p, gfeat], axis=2)

    posb = pos.reshape(B, N_PER, 3)
    idxs, sp3 = _fps_pallas(posb)

    sx = jnp.take_along_axis(x, idxs[:, :, None], axis=1)
    nbr = _knn_pallas(sp3)

    offs = (jnp.arange(B) * SAMPLE)[:, None]
    new_src = (nbr.reshape(B, -1) + offs).reshape(-1)

    sxf = sx.reshape(B * SAMPLE, 192)
    t = sxf @ params['theta_W']
    tmax = _gather_max_sc(t, new_src.astype(jnp.int32))
    resid = sxf @ (params['phi_W'] - params['theta_W']) + (
        params['theta_b'] + params['phi_b'])
    x_out = tmax + resid
    new_dst = (jnp.tile(jnp.repeat(jnp.arange(SAMPLE), KNN)[None, :],
                        (B, 1)) + offs).reshape(-1)
    return (x_out, jnp.stack([new_src, new_dst]))
